# baseline (speedup 1.0000x reference)
import functools

import jax
import jax.numpy as jnp
from jax import lax
from jax.experimental import pallas as pl
from jax.experimental.pallas import tpu as pltpu
from jax.experimental.pallas import tpu_sc as plsc

D_MODEL = 64
SCALE = 8.0

NC = 2
NS = 16
NW = NC * NS
TB = 128
NTR = D_MODEL // 8
RS1 = 65
RS2 = 130
NB1 = 3
NB2 = 4


@functools.lru_cache(maxsize=None)
def _fmt_call(V: int):
    ntiles = V // TB
    tail = V - ntiles * TB
    mesh = plsc.VectorSubcoreMesh(core_axis_name="c", subcore_axis_name="s")

    scratch = (
        [pltpu.VMEM((D_MODEL, TB), jnp.float32) for _ in range(NB1)]
        + [pltpu.VMEM((TB * RS1,), jnp.float32)]
        + [pltpu.VMEM((TB * D_MODEL,), jnp.float32) for _ in range(NB1)]
        + [pltpu.VMEM((tail * D_MODEL,), jnp.float32)]
        + [pltpu.SemaphoreType.DMA for _ in range(2 * NB1 + 1)]
    )

    @functools.partial(
        pl.kernel,
        mesh=mesh,
        out_type=jax.ShapeDtypeStruct((V * D_MODEL,), jnp.float32),
        scratch_types=scratch,
        compiler_params=pltpu.CompilerParams(
            use_tc_tiling_on_sc=True, needs_layout_passes=False),
    )
    def fmt(tt_hbm, tail_hbm, out_hbm, *rest):
        ibuf = rest[:NB1]
        pbuf = rest[NB1]
        cbuf = rest[NB1 + 1:2 * NB1 + 1]
        tailv = rest[2 * NB1 + 1]
        gsem = rest[2 * NB1 + 2:3 * NB1 + 2]
        ssem = rest[3 * NB1 + 2:4 * NB1 + 2]
        tsem = rest[4 * NB1 + 2]

        wid = lax.axis_index("s") * NC + lax.axis_index("c")
        extra = ntiles % NW
        n_mine = ntiles // NW + jnp.where(wid < extra, 1, 0)

        lane = lax.iota(jnp.int32, 16)
        lane_rs = lane * RS1

        @pl.when(wid == NW - 1)
        def _():
            pltpu.async_copy(tail_hbm, tailv, tsem).wait()
            pltpu.async_copy(
                tailv, out_hbm.at[pl.ds(ntiles * TB * D_MODEL,
                                        tail * D_MODEL)], tsem).wait()

        def chunk_of(k):
            return wid + k * NW

        def start_in(k, b):
            c = chunk_of(k)
            pltpu.async_copy(
                tt_hbm.at[pl.ds(0, D_MODEL), pl.ds(c * TB, TB)],
                ibuf[b], gsem[b])

        def wait_in(k, b):
            c = chunk_of(k)
            pltpu.make_async_copy(
                tt_hbm.at[pl.ds(0, D_MODEL), pl.ds(c * TB, TB)],
                ibuf[b], gsem[b]).wait()

        def start_out(k, b):
            c = chunk_of(k)
            pltpu.async_copy(
                cbuf[b], out_hbm.at[pl.ds(c * TB * D_MODEL, TB * D_MODEL)],
                ssem[b])

        def wait_out(k, b):
            c = chunk_of(k)
            pltpu.make_async_copy(
                cbuf[b], out_hbm.at[pl.ds(c * TB * D_MODEL, TB * D_MODEL)],
                ssem[b]).wait()

        for b in range(NB1):
            @pl.when(b < n_mine)
            def _():
                start_in(b, b)

        def chunk_body(k, carry):
            b = lax.rem(k, NB1)

            def do(b):
                wait_in(k, b)

                def tp_body(s8, c):
                    base = lane_rs + s8 * 16 * RS1
                    for d in range(D_MODEL):
                        v = ibuf[b][d, pl.ds(s8 * 16, 16)] * SCALE
                        plsc.store_scatter(pbuf, [base + d], v)
                    return c

                lax.fori_loop(0, TB // 16, tp_body, 0)

                @pl.when(k >= NB1)
                def _():
                    wait_out(k - NB1, b)

                def cp_body(si, c):
                    for q in range(D_MODEL // 16):
                        cbuf[b][pl.ds(si * D_MODEL + q * 16, 16)] = (
                            pbuf[pl.ds(si * RS1 + q * 16, 16)])
                    return c

                lax.fori_loop(0, TB, cp_body, 0, unroll=4)

                @pl.when(k + NB1 < n_mine)
                def _():
                    start_in(k + NB1, b)

                start_out(k, b)

            for bb in range(NB1):
                @pl.when(b == bb)
                def _():
                    do(bb)
            return carry

        lax.fori_loop(0, n_mine, chunk_body, 0)

        def drain(k, carry):
            b = lax.rem(k, NB1)
            for bb in range(NB1):
                @pl.when(b == bb)
                def _():
                    wait_out(k, bb)
            return carry

        lax.fori_loop(jnp.maximum(n_mine - NB1, 0), n_mine, drain, 0)

    return fmt


@functools.lru_cache(maxsize=None)
def _lookup_call(S: int, P: int, V: int):
    mesh = plsc.VectorSubcoreMesh(core_axis_name="c", subcore_axis_name="s")
    n_rounds = P // NB2
    blk_rows = 8

    scratch = (
        [pltpu.VMEM((P, TB), jnp.int32)]
        + [pltpu.VMEM((TB, D_MODEL), jnp.float32) for _ in range(NB2)]
        + [pltpu.VMEM((D_MODEL, RS2), jnp.float32) for _ in range(NB2)]
        + [pltpu.SemaphoreType.DMA for _ in range(2 * NB2)]
    )

    @functools.partial(
        pl.kernel,
        mesh=mesh,
        out_type=jax.ShapeDtypeStruct((P * NTR * NW * blk_rows, TB),
                                      jnp.float32),
        scratch_types=scratch,
        compiler_params=pltpu.CompilerParams(
            use_tc_tiling_on_sc=False, needs_layout_passes=False),
    )
    def lkp(xt_hbm, tab_hbm, out_hbm, idx_v, *rest):
        gbuf = rest[:NB2]
        tbuf = rest[NB2:2 * NB2]
        gsem = rest[2 * NB2:3 * NB2]
        ssem = rest[3 * NB2:4 * NB2]

        wid = lax.axis_index("s") * NC + lax.axis_index("c")
        pltpu.sync_copy(xt_hbm.at[pl.ds(0, P), pl.ds(wid * TB, TB)], idx_v)

        lane = lax.iota(jnp.int32, 16)
        rowv = [lane + q * 16 for q in range(D_MODEL // 16)]

        def start_gather(p, b):
            pltpu.async_copy(tab_hbm.at[idx_v.at[p]], gbuf[b], gsem[b])

        def wait_gather(p, b):
            pltpu.make_async_copy(
                tab_hbm.at[idx_v.at[p]], gbuf[b], gsem[b]).wait()

        def row0(p, tr):
            return ((p * NTR + tr) * NW + wid) * blk_rows

        def start_store(p, b):
            for tr in range(NTR):
                pltpu.async_copy(
                    tbuf[b].at[pl.ds(tr * 8, 8), pl.ds(0, TB)],
                    out_hbm.at[pl.ds(row0(p, tr), 8)],
                    ssem[b])

        def wait_store(p, b):
            for tr in range(NTR):
                pltpu.make_async_copy(
                    tbuf[b].at[pl.ds(tr * 8, 8), pl.ds(0, TB)],
                    out_hbm.at[pl.ds(row0(p, tr), 8)],
                    ssem[b]).wait()

        for b in range(NB2):
            start_gather(b, b)

        def round_body(g, carry):
            for b in range(NB2):
                p = g * NB2 + b
                wait_gather(p, b)

                @pl.when(g > 0)
                def _():
                    wait_store(p - NB2, b)

                def tok_body(si, c):
                    colv = jnp.zeros((16,), jnp.int32) + si
                    for q in range(D_MODEL // 16):
                        v = gbuf[b][si, pl.ds(q * 16, 16)]
                        plsc.store_scatter(tbuf[b], [rowv[q], colv], v)
                    return c

                lax.fori_loop(0, TB, tok_body, 0, unroll=2)

                @pl.when(p + NB2 < P)
                def _():
                    start_gather(p + NB2, b)

                start_store(p, b)
            return carry

        lax.fori_loop(0, n_rounds, round_body, 0)

        for b in range(NB2):
            wait_store((n_rounds - 1) * NB2 + b, b)

    return lkp


def kernel(x, table):
    S, P = x.shape
    V = table.shape[0]
    xt = jnp.transpose(x.astype(jnp.int32))
    tt = jnp.transpose(table)
    ntiles = V // TB
    tailf = (table[ntiles * TB:] * SCALE).reshape(-1)
    lin = _fmt_call(V)(tt, tailf)
    out = _lookup_call(S, P, V)(xt, lin.reshape(V, D_MODEL))
    out = out.reshape(P, NTR, NW, 8, TB)
    return out.transpose(2, 4, 0, 1, 3).reshape(S, P, D_MODEL)


# speedup vs baseline: 1.3295x; 1.3295x over previous
import functools

import jax
import jax.numpy as jnp
from jax import lax
from jax.experimental import pallas as pl
from jax.experimental.pallas import tpu as pltpu
from jax.experimental.pallas import tpu_sc as plsc

D_MODEL = 64
SCALE = 8.0

NC = 2
NS = 16
NW = NC * NS
TB = 128
NTR = D_MODEL // 8
NB1 = 3
NB2 = 4


@functools.lru_cache(maxsize=None)
def _fmt_call(V: int):
    ntiles = V // TB
    tail = V - ntiles * TB
    mesh = plsc.VectorSubcoreMesh(core_axis_name="c", subcore_axis_name="s")

    scratch = (
        [pltpu.VMEM((D_MODEL, TB), jnp.float32) for _ in range(NB1)]
        + [pltpu.VMEM((TB * D_MODEL,), jnp.float32) for _ in range(NB1)]
        + [pltpu.VMEM((tail * D_MODEL,), jnp.float32)]
        + [pltpu.SemaphoreType.DMA for _ in range(2 * NB1 + 1)]
    )

    @functools.partial(
        pl.kernel,
        mesh=mesh,
        out_type=jax.ShapeDtypeStruct((V * D_MODEL,), jnp.float32),
        scratch_types=scratch,
        compiler_params=pltpu.CompilerParams(
            use_tc_tiling_on_sc=True, needs_layout_passes=False),
    )
    def fmt(tt_hbm, tail_hbm, out_hbm, *rest):
        ibuf = rest[:NB1]
        cbuf = rest[NB1:2 * NB1]
        tailv = rest[2 * NB1]
        gsem = rest[2 * NB1 + 1:3 * NB1 + 1]
        ssem = rest[3 * NB1 + 1:4 * NB1 + 1]
        tsem = rest[4 * NB1 + 1]

        wid = lax.axis_index("s") * NC + lax.axis_index("c")
        extra = ntiles % NW
        n_mine = ntiles // NW + jnp.where(wid < extra, 1, 0)

        lane = lax.iota(jnp.int32, 16)
        lane64 = lane * D_MODEL

        @pl.when(wid == NW - 1)
        def _():
            pltpu.async_copy(tail_hbm, tailv, tsem).wait()
            pltpu.async_copy(
                tailv, out_hbm.at[pl.ds(ntiles * TB * D_MODEL,
                                        tail * D_MODEL)], tsem).wait()

        def chunk_of(k):
            return wid + k * NW

        def start_in(k, b):
            c = chunk_of(k)
            pltpu.async_copy(
                tt_hbm.at[pl.ds(0, D_MODEL), pl.ds(c * TB, TB)],
                ibuf[b], gsem[b])

        def wait_in(k, b):
            c = chunk_of(k)
            pltpu.make_async_copy(
                tt_hbm.at[pl.ds(0, D_MODEL), pl.ds(c * TB, TB)],
                ibuf[b], gsem[b]).wait()

        def start_out(k, b):
            c = chunk_of(k)
            pltpu.async_copy(
                cbuf[b], out_hbm.at[pl.ds(c * TB * D_MODEL, TB * D_MODEL)],
                ssem[b])

        def wait_out(k, b):
            c = chunk_of(k)
            pltpu.make_async_copy(
                cbuf[b], out_hbm.at[pl.ds(c * TB * D_MODEL, TB * D_MODEL)],
                ssem[b]).wait()

        for b in range(NB1):
            @pl.when(b < n_mine)
            def _():
                start_in(b, b)

        def chunk_body(k, carry):
            b = lax.rem(k, NB1)

            def do(b):
                wait_in(k, b)

                @pl.when(k >= NB1)
                def _():
                    wait_out(k - NB1, b)

                def tp_body(s8, c):
                    base = lane64 + s8 * (16 * D_MODEL)
                    rotb = lane + s8 * 16
                    for d in range(D_MODEL):
                        v = ibuf[b][d, pl.ds(s8 * 16, 16)] * SCALE
                        rot = (rotb + d) & (D_MODEL - 1)
                        plsc.store_scatter(cbuf[b], [base + rot], v)
                    return c

                lax.fori_loop(0, TB // 16, tp_body, 0)

                @pl.when(k + NB1 < n_mine)
                def _():
                    start_in(k + NB1, b)

                start_out(k, b)

            for bb in range(NB1):
                @pl.when(b == bb)
                def _():
                    do(bb)
            return carry

        lax.fori_loop(0, n_mine, chunk_body, 0)

        def drain(k, carry):
            b = lax.rem(k, NB1)
            for bb in range(NB1):
                @pl.when(b == bb)
                def _():
                    wait_out(k, bb)
            return carry

        lax.fori_loop(jnp.maximum(n_mine - NB1, 0), n_mine, drain, 0)

    return fmt


@functools.lru_cache(maxsize=None)
def _lookup_call(S: int, P: int, V: int):
    mesh = plsc.VectorSubcoreMesh(core_axis_name="c", subcore_axis_name="s")
    n_rounds = P // NB2

    scratch = (
        [pltpu.VMEM((P, TB), jnp.int32)]
        + [pltpu.VMEM((TB, D_MODEL), jnp.float32) for _ in range(NB2)]
        + [pltpu.VMEM((D_MODEL, TB), jnp.float32) for _ in range(NB2)]
        + [pltpu.SemaphoreType.DMA for _ in range(2 * NB2)]
    )

    @functools.partial(
        pl.kernel,
        mesh=mesh,
        out_type=jax.ShapeDtypeStruct((P * NTR * NW * 8, TB), jnp.float32),
        scratch_types=scratch,
        compiler_params=pltpu.CompilerParams(
            use_tc_tiling_on_sc=False, needs_layout_passes=False),
    )
    def lkp(xt_hbm, tab_hbm, out_hbm, idx_v, *rest):
        gbuf = rest[:NB2]
        tbuf = rest[NB2:2 * NB2]
        gsem = rest[2 * NB2:3 * NB2]
        ssem = rest[3 * NB2:4 * NB2]

        wid = lax.axis_index("s") * NC + lax.axis_index("c")
        pltpu.sync_copy(xt_hbm.at[pl.ds(0, P), pl.ds(wid * TB, TB)], idx_v)

        lane = lax.iota(jnp.int32, 16)

        def start_gather(p, b):
            pltpu.async_copy(tab_hbm.at[idx_v.at[p]], gbuf[b], gsem[b])

        def wait_gather(p, b):
            pltpu.make_async_copy(
                tab_hbm.at[idx_v.at[p]], gbuf[b], gsem[b]).wait()

        def row0(p, tr):
            return ((p * NTR + tr) * NW + wid) * 8

        def start_store(p, b):
            for tr in range(NTR):
                pltpu.async_copy(
                    tbuf[b].at[pl.ds(tr * 8, 8)],
                    out_hbm.at[pl.ds(row0(p, tr), 8)],
                    ssem[b])

        def wait_store(p, b):
            for tr in range(NTR):
                pltpu.make_async_copy(
                    tbuf[b].at[pl.ds(tr * 8, 8)],
                    out_hbm.at[pl.ds(row0(p, tr), 8)],
                    ssem[b]).wait()

        for b in range(NB2):
            start_gather(b, b)

        def round_body(g, carry):
            for b in range(NB2):
                p = g * NB2 + b
                wait_gather(p, b)

                @pl.when(g > 0)
                def _():
                    wait_store(p - NB2, b)

                def tok_body(s8, c):
                    sl = pl.ds(s8 * 16, 16)
                    vvec = idx_v[p, sl]
                    rowc = lane + s8 * 16
                    for d in range(D_MODEL):
                        col = (vvec + d) & (D_MODEL - 1)
                        v = plsc.load_gather(gbuf[b], [rowc, col])
                        tbuf[b][d, sl] = v
                    return c

                lax.fori_loop(0, TB // 16, tok_body, 0)

                @pl.when(p + NB2 < P)
                def _():
                    start_gather(p + NB2, b)

                start_store(p, b)
            return carry

        lax.fori_loop(0, n_rounds, round_body, 0)

        for b in range(NB2):
            wait_store((n_rounds - 1) * NB2 + b, b)

    return lkp


def kernel(x, table):
    S, P = x.shape
    V = table.shape[0]
    xt = jnp.transpose(x.astype(jnp.int32))
    tt = jnp.transpose(table)
    ntiles = V // TB
    t_tail = table[ntiles * TB:] * SCALE
    n_tail = t_tail.shape[0]
    j = jnp.arange(D_MODEL)[None, :]
    i = jnp.arange(n_tail)[:, None]
    tailf = jnp.take_along_axis(
        t_tail, (j - i) % D_MODEL, axis=1).reshape(-1)
    lin = _fmt_call(V)(tt, tailf)
    out = _lookup_call(S, P, V)(xt, lin.reshape(V, D_MODEL))
    out = out.reshape(P, NTR, NW, 8, TB)
    return out.transpose(2, 4, 0, 1, 3).reshape(S, P, D_MODEL)
